# Initial kernel scaffold; baseline (speedup 1.0000x reference)
import jax
import jax.numpy as jnp
from jax.experimental import pallas as pl


def kernel(x, edge_index, Wl0, bl0, Wr0, Wl1, bl1, Wr1, Wl2, bl2, Wr2, Wl3, bl3, Wr3, Wl4, bl4, Wr4):
    raise NotImplementedError("write your pallas kernel here")


# speedup vs baseline: 2.2315x; 2.2315x over previous
import functools

import jax
import jax.numpy as jnp
from jax import lax
from jax.experimental import pallas as pl
from jax.experimental.pallas import tpu as pltpu
from jax.experimental.pallas import tpu_sc as plsc

N = 10000
E = 100000
EP = 102400
ROWS = EP // 128
NP = 10240
BN = 1000
NB = N // BN
NSTRIPE = NP // 16
F32 = jnp.float32


def _make_agg(C, W, split):
    n_g = ROWS // 32 if split else ROWS // 16
    npass = C if split else C // 2
    out_rows = (2 if split else 1) * C * NP
    mesh = plsc.VectorSubcoreMesh(core_axis_name="c", subcore_axis_name="s")

    @functools.partial(
        pl.kernel,
        out_type=jax.ShapeDtypeStruct((out_rows, W), F32),
        mesh=mesh,
        scratch_types=[
            pltpu.VMEM((n_g * 128,), jnp.int32),
            pltpu.VMEM((n_g * 128,), jnp.int32),
            pltpu.VMEM((2 * n_g, 64), jnp.int32),
            pltpu.VMEM((64, W), F32),
            pltpu.VMEM((64, W), F32),
            pltpu.VMEM_SHARED((NP, W), F32),
            pltpu.SemaphoreType.DMA,
        ],
    )
    def agg(x_hbm, src_hbm, dst_hbm, out_hbm,
            src_v, adj_v, dst_v, buf, zbuf, acc, sem):
        ci = lax.axis_index("c")
        si = lax.axis_index("s")
        row0 = ci * (ROWS // 2) + si * n_g if split else si * n_g

        pltpu.sync_copy(src_hbm.at[pl.ds(row0 * 128, n_g * 128)], src_v)
        pltpu.sync_copy(dst_hbm.at[pl.ds(row0 * 128, n_g * 128)], adj_v)

        @pl.loop(0, 2 * n_g)
        def _(r):
            @pl.loop(0, 64, step=16)
            def _(cc):
                dst_v[r, pl.ds(cc, 16)] = adj_v[pl.ds(r * 64 + cc, 16)]

        zvec = jnp.zeros((16,), F32)

        @pl.loop(0, 64)
        def _(r):
            @pl.loop(0, W, step=16)
            def _(cc):
                zbuf[r, pl.ds(cc, 16)] = zvec

        for k in range(npass):
            if split:
                base = k * N
            else:
                base = (ci * npass + k) * N
            for j in range(10):
                pltpu.sync_copy(zbuf, acc.at[pl.ds(si * NSTRIPE + j * 64, 64)])
            plsc.subcore_barrier()

            off = jnp.zeros((16,), jnp.int32) + base

            @pl.loop(0, n_g * 128, step=16)
            def _(i):
                adj_v[pl.ds(i, 16)] = src_v[pl.ds(i, 16)] + off

            @pl.loop(0, 2 * n_g)
            def _(g):
                pltpu.async_copy(
                    x_hbm.at[adj_v.at[pl.ds(g * 64, 64)]], buf, sem
                ).wait()
                pltpu.sync_copy(buf, acc.at[dst_v.at[g]], add=True)

            plsc.subcore_barrier()
            if split:
                ob = (ci * C + k) * NP + si * NSTRIPE
            else:
                ob = (ci * npass + k) * NP + si * NSTRIPE
            pltpu.sync_copy(acc.at[pl.ds(si * NSTRIPE, NSTRIPE)],
                            out_hbm.at[pl.ds(ob, NSTRIPE)])

    return agg


def _mm(h, w):
    din, dout = w.shape

    def body(h_ref, w_ref, o_ref):
        o_ref[...] = jnp.dot(h_ref[...], w_ref[...], preferred_element_type=F32,
                         precision=lax.Precision.HIGHEST)

    return pl.pallas_call(
        body,
        grid=(NB,),
        in_specs=[
            pl.BlockSpec((BN, din), lambda n: (n, 0)),
            pl.BlockSpec((din, dout), lambda n: (0, 0)),
        ],
        out_specs=pl.BlockSpec((BN, dout), lambda n: (n, 0)),
        out_shape=jax.ShapeDtypeStruct((N, dout), F32),
    )(h, w)


def _mm_chunk(h, w, C):
    din = w.shape[0]

    def body(h_ref, w_ref, o_ref):
        o_ref[...] = jnp.dot(h_ref[...], w_ref[...], preferred_element_type=F32,
                         precision=lax.Precision.HIGHEST)

    return pl.pallas_call(
        body,
        grid=(NB, C),
        in_specs=[
            pl.BlockSpec((BN, din), lambda n, c: (n, 0)),
            pl.BlockSpec((din, 128), lambda n, c: (0, c)),
        ],
        out_specs=pl.BlockSpec((BN, 128), lambda n, c: (c * NB + n, 0)),
        out_shape=jax.ShapeDtypeStruct((C * N, 128), F32),
    )(h, w)


def _chunk_copy(x, C):
    def body(x_ref, o_ref):
        o_ref[...] = x_ref[...]

    return pl.pallas_call(
        body,
        grid=(NB, C),
        in_specs=[pl.BlockSpec((BN, 128), lambda n, c: (n, c))],
        out_specs=pl.BlockSpec((BN, 128), lambda n, c: (c * NB + n, 0)),
        out_shape=jax.ShapeDtypeStruct((C * N, 128), F32),
    )(x)


def _combine(s, cnt2, b, y, C, W, split, out_w, relu):
    if split:
        s = s.reshape(2, C, NP, W)
        s_spec = pl.BlockSpec((2, C, BN, W), lambda n: (0, 0, n, 0))
    else:
        s = s.reshape(C, NP, W)
        s_spec = pl.BlockSpec((C, BN, W), lambda n: (0, n, 0))

    def body(s_ref, c_ref, b_ref, y_ref, o_ref):
        cnt = c_ref[0, :, 0:1] + c_ref[1, :, 0:1]
        inv = 1.0 / jnp.maximum(cnt, 1.0)
        if split:
            parts = [s_ref[0, c] + s_ref[1, c] for c in range(C)]
        else:
            parts = [s_ref[c] for c in range(C)]
        full = parts[0] if C == 1 else jnp.concatenate(parts, axis=1)
        res = full[:, :out_w] * inv + b_ref[0:1, :] + y_ref[...]
        if relu:
            res = jnp.maximum(res, 0.0)
        o_ref[...] = res

    return pl.pallas_call(
        body,
        grid=(NB,),
        in_specs=[
            s_spec,
            pl.BlockSpec((2, BN, 128), lambda n: (0, n, 0)),
            pl.BlockSpec((1, out_w), lambda n: (0, 0)),
            pl.BlockSpec((BN, out_w), lambda n: (n, 0)),
        ],
        out_specs=pl.BlockSpec((BN, out_w), lambda n: (n, 0)),
        out_shape=jax.ShapeDtypeStruct((N, out_w), F32),
    )(s, cnt2, b, y)


def _l0_combine(s, cnt2, wl, b, y):
    s = s.reshape(6, NP, 128)
    dout = wl.shape[1]

    def body(s_ref, c_ref, w_ref, b_ref, y_ref, o_ref):
        cnt = c_ref[0, :, 0:1] + c_ref[1, :, 0:1]
        inv = 1.0 / jnp.maximum(cnt, 1.0)
        acc = y_ref[...] + b_ref[0:1, :]
        for c in range(6):
            acc = acc + jnp.dot(s_ref[c] * inv, w_ref[c * 128:(c + 1) * 128, :],
                                preferred_element_type=F32,
                                precision=lax.Precision.HIGHEST)
        o_ref[...] = jnp.maximum(acc, 0.0)

    return pl.pallas_call(
        body,
        grid=(NB,),
        in_specs=[
            pl.BlockSpec((6, BN, 128), lambda n: (0, n, 0)),
            pl.BlockSpec((2, BN, 128), lambda n: (0, n, 0)),
            pl.BlockSpec((768, dout), lambda n: (0, 0)),
            pl.BlockSpec((1, dout), lambda n: (0, 0)),
            pl.BlockSpec((BN, dout), lambda n: (n, 0)),
        ],
        out_specs=pl.BlockSpec((BN, dout), lambda n: (n, 0)),
        out_shape=jax.ShapeDtypeStruct((N, dout), F32),
    )(s, cnt2, wl, b, y)


def kernel(x, edge_index, Wl0, bl0, Wr0, Wl1, bl1, Wr1, Wl2, bl2, Wr2,
           Wl3, bl3, Wr3, Wl4, bl4, Wr4):
    src = edge_index[0]
    dst = edge_index[1]
    srcp = jnp.concatenate([src, jnp.zeros((EP - E,), jnp.int32)])
    dstp = jnp.concatenate([dst, jnp.full((EP - E,), N, jnp.int32)])

    agg6 = _make_agg(6, 128, False)
    agg3s = _make_agg(3, 128, True)
    agg1s = _make_agg(1, 128, True)

    ones_tab = jnp.ones((N, 128), F32)
    cnt2 = agg1s(ones_tab, srcp, dstp)
    cnt2 = cnt2.reshape(2, NP, 128)

    b0 = bl0.reshape(1, -1)
    b1 = bl1.reshape(1, -1)
    b2 = bl2.reshape(1, -1)
    b3 = bl3.reshape(1, -1)
    b4 = bl4.reshape(1, -1)

    xc = _chunk_copy(x, 6)
    s0 = agg6(xc, srcp, dstp)
    y0 = _mm(x, Wr0)
    h1 = _l0_combine(s0, cnt2, Wl0, b0, y0)

    p1 = _mm_chunk(h1, Wl1, 6)
    y1 = _mm(h1, Wr1)
    s1 = agg6(p1, srcp, dstp)
    h2 = _combine(s1, cnt2, b1, y1, 6, 128, False, 768, True)

    p2 = _mm_chunk(h2, Wl2, 3)
    y2 = _mm(h2, Wr2)
    s2 = agg3s(p2, srcp, dstp)
    h3 = _combine(s2, cnt2, b2, y2, 3, 128, True, 384, True)

    p3 = _mm(h3, Wl3)
    y3 = _mm(h3, Wr3)
    s3 = agg1s(p3, srcp, dstp)
    h4 = _combine(s3, cnt2, b3, y3, 1, 128, True, 128, True)

    wl4p = jnp.concatenate([Wl4, jnp.zeros((128, 123), F32)], axis=1)
    p4 = _mm(h4, wl4p)
    y4 = _mm(h4, Wr4)
    s4 = agg1s(p4, srcp, dstp)
    out = _combine(s4, cnt2, b4, y4, 1, 128, True, 5, False)

    return (h4, out)


# speedup vs baseline: 2.4665x; 1.1053x over previous
import functools

import jax
import jax.numpy as jnp
from jax import lax
from jax.experimental import pallas as pl
from jax.experimental.pallas import tpu as pltpu
from jax.experimental.pallas import tpu_sc as plsc

N = 10000
E = 100000
EP = 102400
ROWS = EP // 128
NP = 10240
BN = 1000
NB = N // BN
NSTRIPE = NP // 16
F32 = jnp.float32


def _make_agg(C, W, split):
    n_g = ROWS // 32 if split else ROWS // 16
    npass = C if split else C // 2
    out_rows = (2 if split else 1) * C * NP
    mesh = plsc.VectorSubcoreMesh(core_axis_name="c", subcore_axis_name="s")

    @functools.partial(
        pl.kernel,
        out_type=jax.ShapeDtypeStruct((out_rows, W), F32),
        mesh=mesh,
        scratch_types=[
            pltpu.VMEM((n_g * 128 + 64,), jnp.int32),
            pltpu.VMEM((2 * n_g, 64), jnp.int32),
            pltpu.VMEM((64, W), F32),
            pltpu.VMEM((64, W), F32),
            pltpu.VMEM_SHARED((NP, W), F32),
            pltpu.SemaphoreType.DMA,
        ],
    )
    def agg(x_hbm, src_hbm, dst_hbm, out_hbm,
            adj_v, dst_v, buf, zbuf, acc, sem):
        ci = lax.axis_index("c")
        si = lax.axis_index("s")
        row0 = ci * (ROWS // 2) + si * n_g if split else si * n_g
        ng2 = 2 * n_g

        pltpu.sync_copy(dst_hbm.at[pl.ds(row0 * 128, n_g * 128)],
                        adj_v.at[pl.ds(0, n_g * 128)])

        @pl.loop(0, ng2)
        def _(r):
            @pl.loop(0, 64, step=16)
            def _(cc):
                dst_v[r, pl.ds(cc, 16)] = adj_v[pl.ds(r * 64 + cc, 16)]

        zvec = jnp.zeros((16,), F32)
        zivec = jnp.zeros((16,), jnp.int32)

        @pl.loop(0, 64)
        def _(r):
            @pl.loop(0, W, step=16)
            def _(cc):
                zbuf[r, pl.ds(cc, 16)] = zvec

        pltpu.sync_copy(src_hbm.at[pl.ds(row0 * 128, n_g * 128)],
                        adj_v.at[pl.ds(0, n_g * 128)])

        @pl.loop(0, 64, step=16)
        def _(i):
            adj_v[pl.ds(n_g * 128 + i, 16)] = zivec

        def adj_add(delta):
            off = jnp.zeros((16,), jnp.int32) + delta

            @pl.loop(0, n_g * 128, step=16)
            def _(i):
                adj_v[pl.ds(i, 16)] = adj_v[pl.ds(i, 16)] + off

        if not split:
            adj_add(ci * npass * N)
        for k in range(npass):
            if k > 0:
                adj_add(N)
            for j in range(10):
                pltpu.sync_copy(zbuf, acc.at[pl.ds(si * NSTRIPE + j * 64, 64)])
            plsc.subcore_barrier()

            @pl.loop(0, ng2)
            def _(g):
                pltpu.async_copy(
                    x_hbm.at[adj_v.at[pl.ds(g * 64, 64)]], buf, sem
                ).wait()
                pltpu.sync_copy(buf, acc.at[dst_v.at[g]], add=True)

            plsc.subcore_barrier()
            if split:
                ob = (ci * C + k) * NP + si * NSTRIPE
            else:
                ob = (ci * npass + k) * NP + si * NSTRIPE
            pltpu.sync_copy(acc.at[pl.ds(si * NSTRIPE, NSTRIPE)],
                            out_hbm.at[pl.ds(ob, NSTRIPE)])

    return agg


def _dot3(a, b):
    ah = a.astype(jnp.bfloat16)
    al = (a - ah.astype(F32)).astype(jnp.bfloat16)
    bh = b.astype(jnp.bfloat16)
    bl = (b - bh.astype(F32)).astype(jnp.bfloat16)
    d = functools.partial(jnp.dot, preferred_element_type=F32)
    return d(ah, bh) + (d(ah, bl) + d(al, bh))

def _mm(h, w):
    din, dout = w.shape

    def body(h_ref, w_ref, o_ref):
        o_ref[...] = _dot3(h_ref[...], w_ref[...])

    return pl.pallas_call(
        body,
        grid=(NB,),
        in_specs=[
            pl.BlockSpec((BN, din), lambda n: (n, 0)),
            pl.BlockSpec((din, dout), lambda n: (0, 0)),
        ],
        out_specs=pl.BlockSpec((BN, dout), lambda n: (n, 0)),
        out_shape=jax.ShapeDtypeStruct((N, dout), F32),
    )(h, w)


def _mm_chunk(h, w, C):
    din = w.shape[0]

    def body(h_ref, w_ref, o_ref):
        o_ref[...] = _dot3(h_ref[...], w_ref[...])

    return pl.pallas_call(
        body,
        grid=(NB, C),
        in_specs=[
            pl.BlockSpec((BN, din), lambda n, c: (n, 0)),
            pl.BlockSpec((din, 128), lambda n, c: (0, c)),
        ],
        out_specs=pl.BlockSpec((BN, 128), lambda n, c: (c * NB + n, 0)),
        out_shape=jax.ShapeDtypeStruct((C * N, 128), F32),
    )(h, w)


def _chunk_copy(x, C):
    def body(x_ref, o_ref):
        o_ref[...] = x_ref[...]

    return pl.pallas_call(
        body,
        grid=(NB, C),
        in_specs=[pl.BlockSpec((BN, 128), lambda n, c: (n, c))],
        out_specs=pl.BlockSpec((BN, 128), lambda n, c: (c * NB + n, 0)),
        out_shape=jax.ShapeDtypeStruct((C * N, 128), F32),
    )(x)


def _combine(s, cnt2, b, y, C, W, split, out_w, relu):
    if split:
        s = s.reshape(2, C, NP, W)
        s_spec = pl.BlockSpec((2, C, BN, W), lambda n: (0, 0, n, 0))
    else:
        s = s.reshape(C, NP, W)
        s_spec = pl.BlockSpec((C, BN, W), lambda n: (0, n, 0))

    def body(s_ref, c_ref, b_ref, y_ref, o_ref):
        cnt = c_ref[0, :, 0:1] + c_ref[1, :, 0:1]
        inv = 1.0 / jnp.maximum(cnt, 1.0)
        if split:
            parts = [s_ref[0, c] + s_ref[1, c] for c in range(C)]
        else:
            parts = [s_ref[c] for c in range(C)]
        full = parts[0] if C == 1 else jnp.concatenate(parts, axis=1)
        res = full[:, :out_w] * inv + b_ref[0:1, :] + y_ref[...]
        if relu:
            res = jnp.maximum(res, 0.0)
        o_ref[...] = res

    return pl.pallas_call(
        body,
        grid=(NB,),
        in_specs=[
            s_spec,
            pl.BlockSpec((2, BN, 128), lambda n: (0, n, 0)),
            pl.BlockSpec((1, out_w), lambda n: (0, 0)),
            pl.BlockSpec((BN, out_w), lambda n: (n, 0)),
        ],
        out_specs=pl.BlockSpec((BN, out_w), lambda n: (n, 0)),
        out_shape=jax.ShapeDtypeStruct((N, out_w), F32),
    )(s, cnt2, b, y)


def _l0_combine(s, cnt2, wl, b, y):
    s = s.reshape(6, NP, 128)
    dout = wl.shape[1]

    def body(s_ref, c_ref, w_ref, b_ref, y_ref, o_ref):
        cnt = c_ref[0, :, 0:1] + c_ref[1, :, 0:1]
        inv = 1.0 / jnp.maximum(cnt, 1.0)
        acc = y_ref[...] + b_ref[0:1, :]
        for c in range(6):
            acc = acc + _dot3(s_ref[c] * inv, w_ref[c * 128:(c + 1) * 128, :])
        o_ref[...] = jnp.maximum(acc, 0.0)

    return pl.pallas_call(
        body,
        grid=(NB,),
        in_specs=[
            pl.BlockSpec((6, BN, 128), lambda n: (0, n, 0)),
            pl.BlockSpec((2, BN, 128), lambda n: (0, n, 0)),
            pl.BlockSpec((768, dout), lambda n: (0, 0)),
            pl.BlockSpec((1, dout), lambda n: (0, 0)),
            pl.BlockSpec((BN, dout), lambda n: (n, 0)),
        ],
        out_specs=pl.BlockSpec((BN, dout), lambda n: (n, 0)),
        out_shape=jax.ShapeDtypeStruct((N, dout), F32),
    )(s, cnt2, wl, b, y)


def kernel(x, edge_index, Wl0, bl0, Wr0, Wl1, bl1, Wr1, Wl2, bl2, Wr2,
           Wl3, bl3, Wr3, Wl4, bl4, Wr4):
    src = edge_index[0]
    dst = edge_index[1]
    srcp = jnp.concatenate([src, jnp.zeros((EP - E,), jnp.int32)])
    dstp = jnp.concatenate([dst, jnp.full((EP - E,), N, jnp.int32)])

    agg6 = _make_agg(6, 128, False)
    agg3s = _make_agg(3, 128, True)
    agg1s = _make_agg(1, 128, True)

    b0 = bl0.reshape(1, -1)
    b1 = bl1.reshape(1, -1)
    b2 = bl2.reshape(1, -1)
    b3 = bl3.reshape(1, -1)
    b4 = bl4.reshape(1, -1)

    xc = _chunk_copy(x, 6)
    s0 = agg6(xc, srcp, dstp)
    y0 = _mm(x, Wr0)
    ones_tab = jnp.ones((N, 128), F32)
    cnt2 = agg1s(ones_tab, srcp, dstp)
    cnt2 = cnt2.reshape(2, NP, 128)
    h1 = _l0_combine(s0, cnt2, Wl0, b0, y0)

    p1 = _mm_chunk(h1, Wl1, 6)
    y1 = _mm(h1, Wr1)
    s1 = agg6(p1, srcp, dstp)
    h2 = _combine(s1, cnt2, b1, y1, 6, 128, False, 768, True)

    p2 = _mm_chunk(h2, Wl2, 3)
    y2 = _mm(h2, Wr2)
    s2 = agg3s(p2, srcp, dstp)
    h3 = _combine(s2, cnt2, b2, y2, 3, 128, True, 384, True)

    p3 = _mm(h3, Wl3)
    y3 = _mm(h3, Wr3)
    s3 = agg1s(p3, srcp, dstp)
    h4 = _combine(s3, cnt2, b3, y3, 1, 128, True, 128, True)

    wl4p = jnp.concatenate([Wl4, jnp.zeros((128, 123), F32)], axis=1)
    p4 = _mm(h4, wl4p)
    y4 = _mm(h4, Wr4)
    s4 = agg1s(p4, srcp, dstp)
    out = _combine(s4, cnt2, b4, y4, 1, 128, True, 5, False)

    return (h4, out)
